# Initial kernel scaffold; baseline (speedup 1.0000x reference)
import jax
import jax.numpy as jnp
from jax.experimental import pallas as pl


def kernel(feats, idx):
    raise NotImplementedError("write your pallas kernel here")


# speedup vs baseline: 2.8901x; 2.8901x over previous
import functools

import jax
import jax.numpy as jnp
from jax import lax
from jax.experimental import pallas as pl
from jax.experimental.pallas import tpu as pltpu
from jax.experimental.pallas import tpu_sc as plsc

N = 100000
M = 800000
D = 64

NUM_CORES = 2
NUM_SUBCORES = 16
NW = NUM_CORES * NUM_SUBCORES
B_PER_W = M // NW
CHUNK = 1000
NCHUNK = B_PER_W // CHUNK


_mesh = plsc.VectorSubcoreMesh(core_axis_name="c", subcore_axis_name="s")


@functools.partial(
    pl.kernel,
    mesh=_mesh,
    compiler_params=pltpu.CompilerParams(use_tc_tiling_on_sc=False),
    out_type=jax.ShapeDtypeStruct((M, D), jnp.float32),
    scratch_types=[
        pltpu.VMEM((CHUNK,), jnp.int32),
        pltpu.VMEM((CHUNK, D), jnp.float32),
        pltpu.SemaphoreType.DMA,
    ],
)
def _sc_gather(feats_hbm, idx_hbm, out_hbm, idx_v, rows_v, sem):
    wid = lax.axis_index("s") * NUM_CORES + lax.axis_index("c")
    base = wid * B_PER_W

    def body(i, _):
        start = base + i * CHUNK
        pltpu.sync_copy(idx_hbm.at[pl.ds(start, CHUNK)], idx_v)
        pltpu.async_copy(feats_hbm.at[idx_v], rows_v, sem).wait()
        pltpu.sync_copy(rows_v, out_hbm.at[pl.ds(start, CHUNK)])
        return 0

    lax.fori_loop(0, NCHUNK, body, 0)


def kernel(feats, idx):
    return _sc_gather(feats, idx.astype(jnp.int32))


# speedup vs baseline: 2.9727x; 1.0286x over previous
import functools

import jax
import jax.numpy as jnp
from jax import lax
from jax.experimental import pallas as pl
from jax.experimental.pallas import tpu as pltpu
from jax.experimental.pallas import tpu_sc as plsc

N = 100000
M = 800000
D = 64

NUM_CORES = 2
NUM_SUBCORES = 16
NW = NUM_CORES * NUM_SUBCORES
B_PER_W = M // NW
CHUNK = 200
NCHUNK = B_PER_W // CHUNK
NBUF = 5
NROUND = NCHUNK // NBUF


_mesh = plsc.VectorSubcoreMesh(core_axis_name="c", subcore_axis_name="s")


@functools.partial(
    pl.kernel,
    mesh=_mesh,
    compiler_params=pltpu.CompilerParams(use_tc_tiling_on_sc=False),
    out_type=jax.ShapeDtypeStruct((M, D), jnp.float32),
    scratch_types=(
        [pltpu.VMEM((B_PER_W,), jnp.int32)]
        + [pltpu.VMEM((CHUNK, D), jnp.float32) for _ in range(NBUF)]
        + [pltpu.SemaphoreType.DMA for _ in range(2 * NBUF)]
    ),
)
def _sc_gather(feats_hbm, idx_hbm, out_hbm, idx_all, *bufs_and_sems):
    rows = bufs_and_sems[:NBUF]
    sgs = bufs_and_sems[NBUF:2 * NBUF]
    sss = bufs_and_sems[2 * NBUF:3 * NBUF]

    wid = lax.axis_index("s") * NUM_CORES + lax.axis_index("c")
    base = wid * B_PER_W

    pltpu.sync_copy(idx_hbm.at[pl.ds(base, B_PER_W)], idx_all)

    def idx_slice(i):
        return idx_all.at[pl.ds(pl.multiple_of(i * CHUNK, 8), CHUNK)]

    def out_slice(i):
        return out_hbm.at[pl.ds(pl.multiple_of(base + i * CHUNK, 8), CHUNK)]

    def g_start(i, b):
        pltpu.async_copy(feats_hbm.at[idx_slice(i)], rows[b], sgs[b])

    def g_wait(i, b):
        pltpu.make_async_copy(feats_hbm.at[idx_slice(i)], rows[b], sgs[b]).wait()

    def s_start(i, b):
        pltpu.async_copy(rows[b], out_slice(i), sss[b])

    def s_wait(i, b):
        pltpu.make_async_copy(rows[b], out_slice(i), sss[b]).wait()

    for b in range(NBUF):
        g_start(b, b)

    def body(j, _):
        i0 = j * NBUF
        for b in range(NBUF):
            g_wait(i0 + b, b)
            s_start(i0 + b, b)
        for b in range(NBUF):
            s_wait(i0 + b, b)
            g_start(i0 + NBUF + b, b)
        return 0

    lax.fori_loop(0, NROUND - 1, body, 0)

    i0 = (NROUND - 1) * NBUF
    for b in range(NBUF):
        g_wait(i0 + b, b)
        s_start(i0 + b, b)
    for b in range(NBUF):
        s_wait(i0 + b, b)


def kernel(feats, idx):
    return _sc_gather(feats, idx.astype(jnp.int32))
